# baseline (speedup 1.0000x reference)
import functools

import jax
import jax.numpy as jnp
import numpy as np
from jax import lax
from jax.experimental import pallas as pl
from jax.experimental.pallas import tpu as pltpu
from jax.experimental.pallas import tpu_sc as plsc

_MU = 0.02
_NBINS = 10
_NROWS = 262144
_NCOLS = 4
_NW = 32
_ROWS_W = _NROWS // _NW
_FLATS_W = _ROWS_W * _NCOLS
_GROUPS = _ROWS_W // 16
_HIST = _NBINS * 16

_EDGES = [np.float32(float(k) / _NBINS) for k in range(1, _NBINS)]

_mesh = plsc.VectorSubcoreMesh(
    core_axis_name="c", subcore_axis_name="s", num_cores=2, num_subcores=16
)
_params = pltpu.CompilerParams(needs_layout_passes=False)


def _wid():
    return lax.axis_index("s") * 2 + lax.axis_index("c")


def _rsqrt(v):
    r = plsc.bitcast(
        jnp.int32(0x5F3759DF) - (plsc.bitcast(v, jnp.int32) >> 1), jnp.float32
    )
    h = jnp.float32(0.5) * v
    for _ in range(3):
        r = r * (jnp.float32(1.5) - h * r * r)
    return r


@functools.partial(
    pl.kernel,
    out_type=(
        jax.ShapeDtypeStruct((_NROWS,), jnp.float32),
        jax.ShapeDtypeStruct((_NROWS,), jnp.float32),
        jax.ShapeDtypeStruct((_NW, 16), jnp.float32),
        jax.ShapeDtypeStruct((_NW, 16), jnp.float32),
    ),
    mesh=_mesh,
    compiler_params=_params,
    scratch_types=[
        pltpu.VMEM((_FLATS_W,), jnp.float32),
        pltpu.VMEM((_FLATS_W,), jnp.float32),
        pltpu.VMEM((_ROWS_W,), jnp.float32),
        pltpu.VMEM((_ROWS_W,), jnp.float32),
        pltpu.VMEM((16,), jnp.float32),
        pltpu.VMEM((16,), jnp.float32),
    ],
)
def _phase1(in_hbm, tgt_hbm, g_hbm, l_hbm, mn_hbm, mx_hbm,
            in_v, tgt_v, g_v, l_v, mn_v, mx_v):
    wid = _wid()
    base = wid * _FLATS_W
    pltpu.sync_copy(in_hbm.at[pl.ds(base, _FLATS_W)], in_v)
    pltpu.sync_copy(tgt_hbm.at[pl.ds(base, _FLATS_W)], tgt_v)
    iota4 = lax.iota(jnp.int32, 16) * 4
    musq = jnp.float32(_MU * _MU)

    def body(i, carry):
        vmin, vmax = carry
        off = i * 64
        g_acc = jnp.zeros((16,), jnp.float32)
        l_acc = jnp.zeros((16,), jnp.float32)
        for c in range(_NCOLS):
            idx = iota4 + (off + c)
            a = plsc.load_gather(in_v, [idx])
            b = plsc.load_gather(tgt_v, [idx])
            d = a - b
            v = d * d + musq
            r = _rsqrt(v)
            l_acc = l_acc + v * r
            g_acc = g_acc + jnp.abs(d) * r
        l_acc = l_acc - jnp.float32(4.0 * _MU)
        g_v[pl.ds(i * 16, 16)] = g_acc
        l_v[pl.ds(i * 16, 16)] = l_acc
        return jnp.minimum(vmin, g_acc), jnp.maximum(vmax, g_acc)

    init = (
        jnp.full((16,), jnp.inf, jnp.float32),
        jnp.full((16,), -jnp.inf, jnp.float32),
    )
    vmin, vmax = lax.fori_loop(0, _GROUPS, body, init)
    mn_v[...] = vmin
    mx_v[...] = vmax
    pltpu.sync_copy(g_v, g_hbm.at[pl.ds(wid * _ROWS_W, _ROWS_W)])
    pltpu.sync_copy(l_v, l_hbm.at[pl.ds(wid * _ROWS_W, _ROWS_W)])
    pltpu.sync_copy(mn_v, mn_hbm.at[wid])
    pltpu.sync_copy(mx_v, mx_hbm.at[wid])


@functools.partial(
    pl.kernel,
    out_type=(
        jax.ShapeDtypeStruct((_NW, _HIST), jnp.float32),
        jax.ShapeDtypeStruct((_NW, _HIST), jnp.float32),
    ),
    mesh=_mesh,
    compiler_params=_params,
    scratch_types=[
        pltpu.VMEM((_ROWS_W,), jnp.float32),
        pltpu.VMEM((_ROWS_W,), jnp.float32),
        pltpu.VMEM((_NW, 16), jnp.float32),
        pltpu.VMEM((_NW, 16), jnp.float32),
        pltpu.VMEM((_HIST,), jnp.float32),
        pltpu.VMEM((_HIST,), jnp.float32),
    ],
)
def _phase2(g_hbm, l_hbm, mn_hbm, mx_hbm, cnt_hbm, ls_hbm,
            g_v, l_v, mn_v, mx_v, cnt_v, ls_v):
    wid = _wid()
    pltpu.sync_copy(g_hbm.at[pl.ds(wid * _ROWS_W, _ROWS_W)], g_v)
    pltpu.sync_copy(l_hbm.at[pl.ds(wid * _ROWS_W, _ROWS_W)], l_v)
    pltpu.sync_copy(mn_hbm, mn_v)
    pltpu.sync_copy(mx_hbm, mx_v)

    zeros = jnp.zeros((16,), jnp.float32)
    for b in range(_NBINS):
        cnt_v[pl.ds(16 * b, 16)] = zeros
        ls_v[pl.ds(16 * b, 16)] = zeros

    vmn = mn_v[0]
    vmx = mx_v[0]
    for w in range(1, _NW):
        vmn = jnp.minimum(vmn, mn_v[w])
        vmx = jnp.maximum(vmx, mx_v[w])
    ones = jnp.ones((16,), jnp.float32)
    rngv = ones * jnp.max(vmx) - ones * jnp.min(vmn)

    iota16 = lax.iota(jnp.int32, 16)

    def body(i, _):
        g = g_v[pl.ds(i * 16, 16)]
        l = l_v[pl.ds(i * 16, 16)]
        gn = g / rngv
        b = jnp.zeros((16,), jnp.int32)
        for e in _EDGES:
            b = b + (gn >= e).astype(jnp.int32)
        idx = b * 16 + iota16
        plsc.addupdate_scatter(cnt_v, [idx], ones)
        plsc.addupdate_scatter(ls_v, [idx], l)
        return 0

    lax.fori_loop(0, _GROUPS, body, 0)
    pltpu.sync_copy(cnt_v, cnt_hbm.at[wid])
    pltpu.sync_copy(ls_v, ls_hbm.at[wid])


@functools.partial(
    pl.kernel,
    out_type=jax.ShapeDtypeStruct((8,), jnp.float32),
    mesh=_mesh,
    compiler_params=_params,
    scratch_types=[
        pltpu.VMEM((_NW, _HIST), jnp.float32),
        pltpu.VMEM((_NW, _HIST), jnp.float32),
        pltpu.VMEM((16,), jnp.float32),
    ],
)
def _phase3(cnt_hbm, ls_hbm, out_hbm, cnt_v, ls_v, res_v):
    wid = _wid()

    @pl.when(wid == 0)
    def _():
        pltpu.sync_copy(cnt_hbm, cnt_v)
        pltpu.sync_copy(ls_hbm, ls_v)
        ones = jnp.ones((16,), jnp.float32)
        zerov = jnp.zeros((16,), jnp.float32)
        tot_v = ones * jnp.float32(_NROWS)
        acc = zerov
        n = zerov
        for b in range(_NBINS):
            cb = jnp.zeros((16,), jnp.float32)
            sb = jnp.zeros((16,), jnp.float32)
            for w in range(_NW):
                cb = cb + cnt_v[w, pl.ds(16 * b, 16)]
                sb = sb + ls_v[w, pl.ds(16 * b, 16)]
            cnt_vv = ones * jnp.sum(cb)
            s_vv = ones * jnp.sum(sb)
            nz = cnt_vv > zerov
            n = n + jnp.where(nz, ones, zerov)
            wb = jnp.where(nz, tot_v / jnp.maximum(cnt_vv, ones), zerov)
            acc = acc + wb * s_vv
        res = (acc / n / tot_v / (ones * jnp.float32(64.0))
               / (ones * jnp.float32(4096.0)))
        res_v[...] = res
        pltpu.sync_copy(res_v.at[pl.ds(0, 8)], out_hbm)


def kernel(input, target):
    xin = input.reshape(-1)
    xtg = target.reshape(-1)
    g, l, mn, mx = _phase1(xin, xtg)
    cnt, ls = _phase2(g, l, mn, mx)
    out = _phase3(cnt, ls)
    return out[0]


# speedup vs baseline: 6.1575x; 6.1575x over previous
import functools

import jax
import jax.numpy as jnp
import numpy as np
from jax import lax
from jax.experimental import pallas as pl
from jax.experimental.pallas import tpu as pltpu
from jax.experimental.pallas import tpu_sc as plsc

_MU = 0.02
_NBINS = 10
_NROWS = 262144
_NCOLS = 4
_NW = 32
_ROWS_W = _NROWS // _NW
_FLATS_W = _ROWS_W * _NCOLS
_GROUPS = _ROWS_W // 16
_HIST = _NBINS * 16

_EDGES = [np.float32(float(k) / _NBINS) for k in range(1, _NBINS)]

_mesh = plsc.VectorSubcoreMesh(
    core_axis_name="c", subcore_axis_name="s", num_cores=2, num_subcores=16
)
_params = pltpu.CompilerParams(needs_layout_passes=False)


def _wid():
    return lax.axis_index("s") * 2 + lax.axis_index("c")


def _rsqrt(v):
    r = plsc.bitcast(
        jnp.int32(0x5F3759DF) - (plsc.bitcast(v, jnp.int32) >> 1), jnp.float32
    )
    h = jnp.float32(0.5) * v
    for _ in range(3):
        r = r * (jnp.float32(1.5) - h * r * r)
    return r


_NTILES = _NROWS // 128
_TILES_W = _NTILES // _NW


@functools.partial(
    pl.kernel,
    out_type=(
        jax.ShapeDtypeStruct((_NROWS,), jnp.float32),
        jax.ShapeDtypeStruct((_NROWS,), jnp.float32),
        jax.ShapeDtypeStruct((_NW, 16), jnp.float32),
        jax.ShapeDtypeStruct((_NW, 16), jnp.float32),
    ),
    mesh=_mesh,
    compiler_params=_params,
    scratch_types=[
        pltpu.VMEM((_TILES_W, _NCOLS, 128), jnp.float32),
        pltpu.VMEM((_TILES_W, _NCOLS, 128), jnp.float32),
        pltpu.VMEM((_ROWS_W,), jnp.float32),
        pltpu.VMEM((_ROWS_W,), jnp.float32),
        pltpu.VMEM((16,), jnp.float32),
        pltpu.VMEM((16,), jnp.float32),
    ],
)
def _phase1(in_hbm, tgt_hbm, g_hbm, l_hbm, mn_hbm, mx_hbm,
            in_v, tgt_v, g_v, l_v, mn_v, mx_v):
    wid = _wid()
    pltpu.sync_copy(in_hbm.at[pl.ds(wid * _TILES_W, _TILES_W)], in_v)
    pltpu.sync_copy(tgt_hbm.at[pl.ds(wid * _TILES_W, _TILES_W)], tgt_v)
    musq = jnp.float32(_MU * _MU)

    def body(t, carry):
        vmin, vmax = carry
        for j in range(8):
            l0 = 16 * j
            g_acc = jnp.zeros((16,), jnp.float32)
            l_acc = jnp.zeros((16,), jnp.float32)
            for c in range(_NCOLS):
                a = in_v[t, c, pl.ds(l0, 16)]
                b = tgt_v[t, c, pl.ds(l0, 16)]
                d = a - b
                v = d * d + musq
                r = _rsqrt(v)
                l_acc = l_acc + v * r
                g_acc = g_acc + jnp.abs(d) * r
            l_acc = l_acc - jnp.float32(4.0 * _MU)
            g_v[pl.ds(t * 128 + l0, 16)] = g_acc
            l_v[pl.ds(t * 128 + l0, 16)] = l_acc
            vmin = jnp.minimum(vmin, g_acc)
            vmax = jnp.maximum(vmax, g_acc)
        return vmin, vmax

    init = (
        jnp.full((16,), jnp.inf, jnp.float32),
        jnp.full((16,), -jnp.inf, jnp.float32),
    )
    vmin, vmax = lax.fori_loop(0, _TILES_W, body, init)
    mn_v[...] = vmin
    mx_v[...] = vmax
    pltpu.sync_copy(g_v, g_hbm.at[pl.ds(wid * _ROWS_W, _ROWS_W)])
    pltpu.sync_copy(l_v, l_hbm.at[pl.ds(wid * _ROWS_W, _ROWS_W)])
    pltpu.sync_copy(mn_v, mn_hbm.at[wid])
    pltpu.sync_copy(mx_v, mx_hbm.at[wid])


@functools.partial(
    pl.kernel,
    out_type=(
        jax.ShapeDtypeStruct((_NW, _HIST), jnp.float32),
        jax.ShapeDtypeStruct((_NW, _HIST), jnp.float32),
    ),
    mesh=_mesh,
    compiler_params=_params,
    scratch_types=[
        pltpu.VMEM((_ROWS_W,), jnp.float32),
        pltpu.VMEM((_ROWS_W,), jnp.float32),
        pltpu.VMEM((_NW, 16), jnp.float32),
        pltpu.VMEM((_NW, 16), jnp.float32),
        pltpu.VMEM((_HIST,), jnp.float32),
        pltpu.VMEM((_HIST,), jnp.float32),
    ],
)
def _phase2(g_hbm, l_hbm, mn_hbm, mx_hbm, cnt_hbm, ls_hbm,
            g_v, l_v, mn_v, mx_v, cnt_v, ls_v):
    wid = _wid()
    pltpu.sync_copy(g_hbm.at[pl.ds(wid * _ROWS_W, _ROWS_W)], g_v)
    pltpu.sync_copy(l_hbm.at[pl.ds(wid * _ROWS_W, _ROWS_W)], l_v)
    pltpu.sync_copy(mn_hbm, mn_v)
    pltpu.sync_copy(mx_hbm, mx_v)

    zeros = jnp.zeros((16,), jnp.float32)
    for b in range(_NBINS):
        cnt_v[pl.ds(16 * b, 16)] = zeros
        ls_v[pl.ds(16 * b, 16)] = zeros

    vmn = mn_v[0]
    vmx = mx_v[0]
    for w in range(1, _NW):
        vmn = jnp.minimum(vmn, mn_v[w])
        vmx = jnp.maximum(vmx, mx_v[w])
    ones = jnp.ones((16,), jnp.float32)
    rngv = ones * jnp.max(vmx) - ones * jnp.min(vmn)

    iota16 = lax.iota(jnp.int32, 16)

    def body(i, _):
        g = g_v[pl.ds(i * 16, 16)]
        l = l_v[pl.ds(i * 16, 16)]
        gn = g / rngv
        b = jnp.zeros((16,), jnp.int32)
        for e in _EDGES:
            b = b + (gn >= e).astype(jnp.int32)
        idx = b * 16 + iota16
        plsc.addupdate_scatter(cnt_v, [idx], ones)
        plsc.addupdate_scatter(ls_v, [idx], l)
        return 0

    lax.fori_loop(0, _GROUPS, body, 0)
    pltpu.sync_copy(cnt_v, cnt_hbm.at[wid])
    pltpu.sync_copy(ls_v, ls_hbm.at[wid])


@functools.partial(
    pl.kernel,
    out_type=jax.ShapeDtypeStruct((8,), jnp.float32),
    mesh=_mesh,
    compiler_params=_params,
    scratch_types=[
        pltpu.VMEM((_NW, _HIST), jnp.float32),
        pltpu.VMEM((_NW, _HIST), jnp.float32),
        pltpu.VMEM((16,), jnp.float32),
    ],
)
def _phase3(cnt_hbm, ls_hbm, out_hbm, cnt_v, ls_v, res_v):
    wid = _wid()

    @pl.when(wid == 0)
    def _():
        pltpu.sync_copy(cnt_hbm, cnt_v)
        pltpu.sync_copy(ls_hbm, ls_v)
        ones = jnp.ones((16,), jnp.float32)
        zerov = jnp.zeros((16,), jnp.float32)
        tot_v = ones * jnp.float32(_NROWS)
        acc = zerov
        n = zerov
        for b in range(_NBINS):
            cb = jnp.zeros((16,), jnp.float32)
            sb = jnp.zeros((16,), jnp.float32)
            for w in range(_NW):
                cb = cb + cnt_v[w, pl.ds(16 * b, 16)]
                sb = sb + ls_v[w, pl.ds(16 * b, 16)]
            cnt_vv = ones * jnp.sum(cb)
            s_vv = ones * jnp.sum(sb)
            nz = cnt_vv > zerov
            n = n + jnp.where(nz, ones, zerov)
            wb = jnp.where(nz, tot_v / jnp.maximum(cnt_vv, ones), zerov)
            acc = acc + wb * s_vv
        res = (acc / n / tot_v / (ones * jnp.float32(64.0))
               / (ones * jnp.float32(4096.0)))
        res_v[...] = res
        pltpu.sync_copy(res_v.at[pl.ds(0, 8)], out_hbm)


def kernel(input, target):
    xin = input.reshape(_NTILES, 128, _NCOLS).transpose(0, 2, 1)
    xtg = target.reshape(_NTILES, 128, _NCOLS).transpose(0, 2, 1)
    g, l, mn, mx = _phase1(xin, xtg)
    cnt, ls = _phase2(g, l, mn, mx)
    out = _phase3(cnt, ls)
    return out[0]


# speedup vs baseline: 6.9761x; 1.1329x over previous
import functools

import jax
import jax.numpy as jnp
import numpy as np
from jax import lax
from jax.experimental import pallas as pl
from jax.experimental.pallas import tpu as pltpu
from jax.experimental.pallas import tpu_sc as plsc

_MU = 0.02
_NBINS = 10
_NROWS = 262144
_NCOLS = 4
_NW = 32
_ROWS_W = _NROWS // _NW
_FLATS_W = _ROWS_W * _NCOLS
_GROUPS = _ROWS_W // 16
_HIST = _NBINS * 16

_EDGES = [np.float32(float(k) / _NBINS) for k in range(1, _NBINS)]

_mesh = plsc.VectorSubcoreMesh(
    core_axis_name="c", subcore_axis_name="s", num_cores=2, num_subcores=16
)
_params = pltpu.CompilerParams(needs_layout_passes=False)


def _wid():
    return lax.axis_index("s") * 2 + lax.axis_index("c")


def _rsqrt(v):
    r = plsc.bitcast(
        jnp.int32(0x5F3759DF) - (plsc.bitcast(v, jnp.int32) >> 1), jnp.float32
    )
    h = jnp.float32(0.5) * v
    for _ in range(3):
        r = r * (jnp.float32(1.5) - h * r * r)
    return r


_NTILES = _NROWS // 128
_TILES_W = _NTILES // _NW


@functools.partial(
    pl.kernel,
    out_type=(
        jax.ShapeDtypeStruct((_NROWS,), jnp.float32),
        jax.ShapeDtypeStruct((_NROWS,), jnp.float32),
        jax.ShapeDtypeStruct((_NW, 16), jnp.float32),
        jax.ShapeDtypeStruct((_NW, 16), jnp.float32),
    ),
    mesh=_mesh,
    compiler_params=_params,
    scratch_types=[
        pltpu.VMEM((_TILES_W, _NCOLS, 128), jnp.float32),
        pltpu.VMEM((_TILES_W, _NCOLS, 128), jnp.float32),
        pltpu.VMEM((_ROWS_W,), jnp.float32),
        pltpu.VMEM((_ROWS_W,), jnp.float32),
        pltpu.VMEM((16,), jnp.float32),
        pltpu.VMEM((16,), jnp.float32),
    ],
)
def _phase1(in_hbm, tgt_hbm, g_hbm, l_hbm, mn_hbm, mx_hbm,
            in_v, tgt_v, g_v, l_v, mn_v, mx_v):
    wid = _wid()
    pltpu.sync_copy(in_hbm.at[pl.ds(wid * _TILES_W, _TILES_W)], in_v)
    pltpu.sync_copy(tgt_hbm.at[pl.ds(wid * _TILES_W, _TILES_W)], tgt_v)
    musq = jnp.float32(_MU * _MU)

    init = (
        jnp.full((16,), jnp.inf, jnp.float32),
        jnp.full((16,), -jnp.inf, jnp.float32),
    )

    @plsc.parallel_loop(0, _TILES_W, unroll=2, carry=init)
    def _loop(t, carry):
        vmin, vmax = carry
        for j in range(8):
            l0 = 16 * j
            g_acc = jnp.zeros((16,), jnp.float32)
            l_acc = jnp.zeros((16,), jnp.float32)
            for c in range(_NCOLS):
                a = in_v[t, c, pl.ds(l0, 16)]
                b = tgt_v[t, c, pl.ds(l0, 16)]
                d = a - b
                v = d * d + musq
                r = _rsqrt(v)
                l_acc = l_acc + v * r
                g_acc = g_acc + jnp.abs(d) * r
            l_acc = l_acc - jnp.float32(4.0 * _MU)
            g_v[pl.ds(t * 128 + l0, 16)] = g_acc
            l_v[pl.ds(t * 128 + l0, 16)] = l_acc
            vmin = jnp.minimum(vmin, g_acc)
            vmax = jnp.maximum(vmax, g_acc)
        return vmin, vmax

    vmin, vmax = _loop
    mn_v[...] = vmin
    mx_v[...] = vmax
    pltpu.sync_copy(g_v, g_hbm.at[pl.ds(wid * _ROWS_W, _ROWS_W)])
    pltpu.sync_copy(l_v, l_hbm.at[pl.ds(wid * _ROWS_W, _ROWS_W)])
    pltpu.sync_copy(mn_v, mn_hbm.at[wid])
    pltpu.sync_copy(mx_v, mx_hbm.at[wid])


@functools.partial(
    pl.kernel,
    out_type=(
        jax.ShapeDtypeStruct((_NW, _HIST), jnp.float32),
        jax.ShapeDtypeStruct((_NW, _HIST), jnp.float32),
    ),
    mesh=_mesh,
    compiler_params=_params,
    scratch_types=[
        pltpu.VMEM((_ROWS_W,), jnp.float32),
        pltpu.VMEM((_ROWS_W,), jnp.float32),
        pltpu.VMEM((_NW, 16), jnp.float32),
        pltpu.VMEM((_NW, 16), jnp.float32),
        pltpu.VMEM((_HIST,), jnp.float32),
        pltpu.VMEM((_HIST,), jnp.float32),
    ],
)
def _phase2(g_hbm, l_hbm, mn_hbm, mx_hbm, cnt_hbm, ls_hbm,
            g_v, l_v, mn_v, mx_v, cnt_v, ls_v):
    wid = _wid()
    pltpu.sync_copy(g_hbm.at[pl.ds(wid * _ROWS_W, _ROWS_W)], g_v)
    pltpu.sync_copy(l_hbm.at[pl.ds(wid * _ROWS_W, _ROWS_W)], l_v)
    pltpu.sync_copy(mn_hbm, mn_v)
    pltpu.sync_copy(mx_hbm, mx_v)

    zeros = jnp.zeros((16,), jnp.float32)
    for b in range(_NBINS):
        cnt_v[pl.ds(16 * b, 16)] = zeros
        ls_v[pl.ds(16 * b, 16)] = zeros

    vmn = mn_v[0]
    vmx = mx_v[0]
    for w in range(1, _NW):
        vmn = jnp.minimum(vmn, mn_v[w])
        vmx = jnp.maximum(vmx, mx_v[w])
    ones = jnp.ones((16,), jnp.float32)
    rngv = ones * jnp.max(vmx) - ones * jnp.min(vmn)

    iota16 = lax.iota(jnp.int32, 16)

    @plsc.parallel_loop(0, _GROUPS, unroll=4)
    def _loop(i):
        g = g_v[pl.ds(i * 16, 16)]
        l = l_v[pl.ds(i * 16, 16)]
        gn = g / rngv
        b = jnp.zeros((16,), jnp.int32)
        for e in _EDGES:
            b = b + (gn >= e).astype(jnp.int32)
        idx = b * 16 + iota16
        plsc.addupdate_scatter(cnt_v, [idx], ones)
        plsc.addupdate_scatter(ls_v, [idx], l)
    pltpu.sync_copy(cnt_v, cnt_hbm.at[wid])
    pltpu.sync_copy(ls_v, ls_hbm.at[wid])


@functools.partial(
    pl.kernel,
    out_type=jax.ShapeDtypeStruct((8,), jnp.float32),
    mesh=_mesh,
    compiler_params=_params,
    scratch_types=[
        pltpu.VMEM((_NW, _HIST), jnp.float32),
        pltpu.VMEM((_NW, _HIST), jnp.float32),
        pltpu.VMEM((16,), jnp.float32),
    ],
)
def _phase3(cnt_hbm, ls_hbm, out_hbm, cnt_v, ls_v, res_v):
    wid = _wid()

    @pl.when(wid == 0)
    def _():
        pltpu.sync_copy(cnt_hbm, cnt_v)
        pltpu.sync_copy(ls_hbm, ls_v)
        ones = jnp.ones((16,), jnp.float32)
        zerov = jnp.zeros((16,), jnp.float32)
        tot_v = ones * jnp.float32(_NROWS)
        acc = zerov
        n = zerov
        for b in range(_NBINS):
            cb = jnp.zeros((16,), jnp.float32)
            sb = jnp.zeros((16,), jnp.float32)
            for w in range(_NW):
                cb = cb + cnt_v[w, pl.ds(16 * b, 16)]
                sb = sb + ls_v[w, pl.ds(16 * b, 16)]
            cnt_vv = ones * jnp.sum(cb)
            s_vv = ones * jnp.sum(sb)
            nz = cnt_vv > zerov
            n = n + jnp.where(nz, ones, zerov)
            wb = jnp.where(nz, tot_v / jnp.maximum(cnt_vv, ones), zerov)
            acc = acc + wb * s_vv
        res = (acc / n / tot_v / (ones * jnp.float32(64.0))
               / (ones * jnp.float32(4096.0)))
        res_v[...] = res
        pltpu.sync_copy(res_v.at[pl.ds(0, 8)], out_hbm)


def kernel(input, target):
    xin = input.reshape(_NTILES, 128, _NCOLS).transpose(0, 2, 1)
    xtg = target.reshape(_NTILES, 128, _NCOLS).transpose(0, 2, 1)
    g, l, mn, mx = _phase1(xin, xtg)
    cnt, ls = _phase2(g, l, mn, mx)
    out = _phase3(cnt, ls)
    return out[0]
